# baseline (speedup 1.0000x reference)
import functools

import jax
import jax.numpy as jnp
import numpy as np
from jax import lax
from jax.experimental import pallas as pl
from jax.experimental.pallas import tpu as pltpu
from jax.experimental.pallas import tpu_sc as plsc

RADIUS2 = 0.15 * 0.15
NDIM = 3
HIDDEN = 64
NF = 64
N_IN = 1024
N_Q = 512
K = 48
NC = 2
NS = 16
NW = NC * NS
QPW = N_Q // NW
L = 16
NCHUNK = N_IN // L


def _prep_body(ypos_ref, xpos_ref, feat_ref, wys_ref, wyc_ref, wxs_ref,
               wxc_ref, wf_ref, b1_ref, freqs_ref, hyh_ref, hyl_ref, hx_ref):
    freqs = freqs_ref[...]
    acc_y = jnp.dot(feat_ref[...], wf_ref[...],
                    preferred_element_type=jnp.float32)
    for d in range(NDIM):
        ph = ypos_ref[:, d:d + 1] * freqs
        acc_y += jnp.dot(jnp.sin(ph), wys_ref[d * NF:(d + 1) * NF, :],
                         preferred_element_type=jnp.float32)
        acc_y += jnp.dot(jnp.cos(ph), wyc_ref[d * NF:(d + 1) * NF, :],
                         preferred_element_type=jnp.float32)
    hi = acc_y.astype(jnp.bfloat16)
    hyh_ref[...] = hi
    hyl_ref[...] = (acc_y - hi.astype(jnp.float32)).astype(jnp.bfloat16)
    acc_x = jnp.broadcast_to(b1_ref[...], (N_Q, 2 * HIDDEN))
    for d in range(NDIM):
        ph = xpos_ref[:, d:d + 1] * freqs
        acc_x = acc_x + jnp.dot(jnp.sin(ph), wxs_ref[d * NF:(d + 1) * NF, :],
                                preferred_element_type=jnp.float32)
        acc_x = acc_x + jnp.dot(jnp.cos(ph), wxc_ref[d * NF:(d + 1) * NF, :],
                                preferred_element_type=jnp.float32)
    hx_ref[...] = acc_x


def _prep(ypos, xpos, feat, wys, wyc, wxs, wxc, wf, b1, freqs, *, interpret=False):
    return pl.pallas_call(
        _prep_body,
        out_shape=(
            jax.ShapeDtypeStruct((N_IN, 2 * HIDDEN), jnp.bfloat16),
            jax.ShapeDtypeStruct((N_IN, 2 * HIDDEN), jnp.bfloat16),
            jax.ShapeDtypeStruct((N_Q, 2 * HIDDEN), jnp.float32),
        ),
        interpret=interpret,
    )(ypos, xpos, feat, wys, wyc, wxs, wxc, wf, b1, freqs)


def _sc_body(ypos_hbm, xpos_hbm, idx_hbm, cnt_hbm,
             yv, qxv, qyv, qzv, idxv, cntv):
    wid = lax.axis_index("s") * NC + lax.axis_index("c")
    qbase = wid * QPW
    pltpu.sync_copy(ypos_hbm, yv)
    pltpu.sync_copy(xpos_hbm.at[0, pl.ds(qbase, QPW)], qxv)
    pltpu.sync_copy(xpos_hbm.at[1, pl.ds(qbase, QPW)], qyv)
    pltpu.sync_copy(xpos_hbm.at[2, pl.ds(qbase, QPW)], qzv)
    lanes = lax.iota(jnp.int32, L)

    def per_query(q, cntvec):
        qi = jnp.full((L,), q, jnp.int32)
        qx = plsc.load_gather(qxv, [qi])
        qy = plsc.load_gather(qyv, [qi])
        qz = plsc.load_gather(qzv, [qi])

        for c in range(K // L):
            plsc.store_scatter(idxv, [qi, lanes + c * L],
                               jnp.zeros((L,), jnp.int32))

        @plsc.parallel_loop(0, NCHUNK, carry=jnp.zeros((L,), jnp.int32),
                            unroll=4)
        def cnt_loop(c, cnt):
            base = c * L
            dx = yv[0, pl.ds(base, L)] - qx
            dy = yv[1, pl.ds(base, L)] - qy
            dz = yv[2, pl.ds(base, L)] - qz
            d2 = dx * dx + dy * dy + dz * dz
            mask = d2 < RADIUS2
            mi = mask.astype(jnp.int32)
            pos = cnt + plsc.cumsum(mi) - 1
            m2 = mask & (pos < K)
            posc = jnp.minimum(jnp.maximum(pos, 0), K - 1)
            plsc.store_scatter(idxv, [qi, posc], lanes + base, mask=m2)
            return cnt + plsc.all_reduce_population_count(mask)

        return jnp.where(lanes == q, cnt_loop, cntvec)

    cntvec = lax.fori_loop(0, QPW, per_query, jnp.zeros((L,), jnp.int32))
    cntv[...] = cntvec
    pltpu.sync_copy(idxv, idx_hbm.at[pl.ds(qbase, QPW)])
    pltpu.sync_copy(cntv, cnt_hbm.at[pl.ds(qbase, QPW)])


def _sc_search(ypos_t, xpos_t):
    mesh = plsc.VectorSubcoreMesh(core_axis_name="c", subcore_axis_name="s")
    k = pl.kernel(
        _sc_body,
        out_type=(
            jax.ShapeDtypeStruct((N_Q, K), jnp.int32),
            jax.ShapeDtypeStruct((N_Q,), jnp.int32),
        ),
        mesh=mesh,
        compiler_params=pltpu.CompilerParams(needs_layout_passes=False),
        scratch_types=[
            pltpu.VMEM((NDIM, N_IN), jnp.float32),
            pltpu.VMEM((QPW,), jnp.float32),
            pltpu.VMEM((QPW,), jnp.float32),
            pltpu.VMEM((QPW,), jnp.float32),
            pltpu.VMEM((QPW, K), jnp.int32),
            pltpu.VMEM((QPW,), jnp.int32),
        ],
    )
    return k(ypos_t, xpos_t)


_BQ = 64


def _mlp_body(idx_ref, hyh_ref, hyl_ref, hx_ref, cnt_ref, w2_ref, b2_ref, o_ref):
    idx = idx_ref[...]
    pio = lax.broadcasted_iota(jnp.int32, (_BQ * K, N_IN), 1)
    p = (idx == pio).astype(jnp.bfloat16)
    g = jnp.dot(p, hyh_ref[...], preferred_element_type=jnp.float32)
    g3 = g.reshape(_BQ, K, 2 * HIDDEN)
    pair = g3 + hx_ref[...][:, None, :]
    act = 0.5 * pair * (1.0 + lax.erf(pair * np.float32(1.0 / np.sqrt(2.0))))
    cnt = cnt_ref[...]
    kio = lax.broadcasted_iota(jnp.int32, (_BQ, K, 2 * HIDDEN), 1)
    masked = jnp.where(kio < cnt.astype(jnp.int32)[:, :, None], act, 0.0)
    summed = jnp.sum(masked, axis=1)
    res = jnp.dot(summed, w2_ref[...], preferred_element_type=jnp.float32)
    res = res / jnp.maximum(cnt, 1.0)
    o_ref[...] = res + b2_ref[...] * (cnt > 0.0).astype(jnp.float32)


def _mlp(idx_flat, hyh, hyl, hx, cnt_f32, w2, b2, *, interpret=False):
    grid = (N_Q // _BQ,)
    return pl.pallas_call(
        _mlp_body,
        grid=grid,
        in_specs=[
            pl.BlockSpec((_BQ * K, 1), lambda i: (i, 0)),
            pl.BlockSpec((N_IN, 2 * HIDDEN), lambda i: (0, 0)),
            pl.BlockSpec((N_IN, 2 * HIDDEN), lambda i: (0, 0)),
            pl.BlockSpec((_BQ, 2 * HIDDEN), lambda i: (i, 0)),
            pl.BlockSpec((_BQ, 1), lambda i: (i, 0)),
            pl.BlockSpec((2 * HIDDEN, HIDDEN), lambda i: (0, 0)),
            pl.BlockSpec((1, HIDDEN), lambda i: (0, 0)),
        ],
        out_specs=pl.BlockSpec((_BQ, HIDDEN), lambda i: (i, 0)),
        out_shape=jax.ShapeDtypeStruct((N_Q, HIDDEN), jnp.float32),
        interpret=interpret,
    )(idx_flat, hyh, hyl, hx, cnt_f32, w2, b2)


_SIN_ROWS = np.repeat(np.arange(NDIM) * 2 * NF, NF) + 2 * np.tile(np.arange(NF), NDIM)
_FREQS = ((1.0 / 10000.0) ** (np.arange(NF, dtype=np.float64) / NF)).astype(np.float32)


def kernel(input_feat, input_pos, query_pos, W1, b1, W2, b2):
    y = input_pos[0]
    x = query_pos[0]
    pos_out = NDIM * NF * 2
    wys = W1[_SIN_ROWS, :]
    wyc = W1[_SIN_ROWS + 1, :]
    wxs = W1[pos_out + _SIN_ROWS, :]
    wxc = W1[pos_out + _SIN_ROWS + 1, :]
    wf = W1[2 * pos_out:, :]
    freqs = jnp.asarray(_FREQS).reshape(1, NF)
    hyh, hyl, hx = _prep(y, x, input_feat, wys, wyc, wxs, wxc, wf,
                         b1.reshape(1, -1), freqs)
    idx, counts = _sc_search(y.T, x.T)
    cnt_f32 = counts.astype(jnp.float32).reshape(N_Q, 1)
    idx_flat = idx.reshape(N_Q * K, 1)
    return _mlp(idx_flat, hyh, hyl, hx, cnt_f32, W2, b2.reshape(1, -1))


# speedup vs baseline: 1.4608x; 1.4608x over previous
import functools

import jax
import jax.numpy as jnp
import numpy as np
from jax import lax
from jax.experimental import pallas as pl
from jax.experimental.pallas import tpu as pltpu
from jax.experimental.pallas import tpu_sc as plsc

RADIUS2 = 0.15 * 0.15
NDIM = 3
HIDDEN = 64
NF = 64
N_IN = 1024
N_Q = 512
K = 48
NC = 2
NS = 16
NW = NC * NS
QPW = N_Q // NW
L = 16
NCHUNK = N_IN // L


def _prep_body(ypos_ref, xpos_ref, feat_ref, w1p_ref, b1_ref, freqs_ref,
               hyh_ref, hx_ref):
    freqs = freqs_ref[...]
    acc_y = jnp.dot(feat_ref[...], w1p_ref[4 * NDIM * NF:, :],
                    preferred_element_type=jnp.float32)
    for d in range(NDIM):
        ph = ypos_ref[:, d:d + 1] * freqs
        acc_y += jnp.dot(jnp.sin(ph), w1p_ref[d * NF:(d + 1) * NF, :],
                         preferred_element_type=jnp.float32)
        acc_y += jnp.dot(jnp.cos(ph),
                         w1p_ref[(NDIM + d) * NF:(NDIM + d + 1) * NF, :],
                         preferred_element_type=jnp.float32)
    hyh_ref[...] = acc_y.astype(jnp.bfloat16)
    acc_x = jnp.broadcast_to(b1_ref[...], (N_Q, 2 * HIDDEN))
    for d in range(NDIM):
        ph = xpos_ref[:, d:d + 1] * freqs
        acc_x = acc_x + jnp.dot(
            jnp.sin(ph), w1p_ref[(2 * NDIM + d) * NF:(2 * NDIM + d + 1) * NF, :],
            preferred_element_type=jnp.float32)
        acc_x = acc_x + jnp.dot(
            jnp.cos(ph), w1p_ref[(3 * NDIM + d) * NF:(3 * NDIM + d + 1) * NF, :],
            preferred_element_type=jnp.float32)
    hx_ref[...] = acc_x


def _prep(ypos, xpos, feat, w1p, b1, freqs, *, interpret=False):
    return pl.pallas_call(
        _prep_body,
        out_shape=(
            jax.ShapeDtypeStruct((N_IN, 2 * HIDDEN), jnp.bfloat16),
            jax.ShapeDtypeStruct((N_Q, 2 * HIDDEN), jnp.float32),
        ),
        interpret=interpret,
    )(ypos, xpos, feat, w1p, b1, freqs)


def _sc_body(ypos_hbm, xpos_hbm, idx_hbm, yv, qv, idxv):
    wid = lax.axis_index("s") * NC + lax.axis_index("c")
    qbase = wid * QPW
    pltpu.sync_copy(ypos_hbm, yv)
    pltpu.sync_copy(xpos_hbm.at[pl.ds(qbase * NDIM, QPW * NDIM)], qv)
    lanes = lax.iota(jnp.int32, L)
    lanes3 = lanes * NDIM

    def per_query(q, _):
        qi = jnp.full((L,), q, jnp.int32)
        q3 = q * NDIM
        qx = plsc.load_gather(qv, [jnp.full((L,), q3, jnp.int32)])
        qy = plsc.load_gather(qv, [jnp.full((L,), q3 + 1, jnp.int32)])
        qz = plsc.load_gather(qv, [jnp.full((L,), q3 + 2, jnp.int32)])

        for c in range(K // L):
            plsc.store_scatter(idxv, [qi, lanes + c * L],
                               jnp.full((L,), -1, jnp.int32))

        @plsc.parallel_loop(0, NCHUNK, carry=jnp.zeros((L,), jnp.int32),
                            unroll=4)
        def cnt_loop(c, cnt):
            b3 = c * (L * NDIM)
            dx = plsc.load_gather(yv, [lanes3 + b3]) - qx
            dy = plsc.load_gather(yv, [lanes3 + (b3 + 1)]) - qy
            dz = plsc.load_gather(yv, [lanes3 + (b3 + 2)]) - qz
            d2 = dx * dx + dy * dy + dz * dz
            mask = d2 < RADIUS2
            mi = mask.astype(jnp.int32)
            pos = cnt + plsc.cumsum(mi) - 1
            m2 = mask & (pos < K)
            posc = jnp.minimum(jnp.maximum(pos, 0), K - 1)
            plsc.store_scatter(idxv, [qi, posc], lanes + c * L, mask=m2)
            return cnt + plsc.all_reduce_population_count(mask)

        return 0

    lax.fori_loop(0, QPW, per_query, 0)
    pltpu.sync_copy(idxv, idx_hbm.at[pl.ds(qbase, QPW)])


def _sc_search(ypos_flat, xpos_flat):
    mesh = plsc.VectorSubcoreMesh(core_axis_name="c", subcore_axis_name="s")
    k = pl.kernel(
        _sc_body,
        out_type=jax.ShapeDtypeStruct((N_Q, K), jnp.int32),
        mesh=mesh,
        compiler_params=pltpu.CompilerParams(needs_layout_passes=False),
        scratch_types=[
            pltpu.VMEM((NDIM * N_IN,), jnp.float32),
            pltpu.VMEM((NDIM * QPW,), jnp.float32),
            pltpu.VMEM((QPW, K), jnp.int32),
        ],
    )
    return k(ypos_flat, xpos_flat)


_BQ = 64


def _mlp_body(idx_ref, hyh_ref, hx_ref, w2_ref, b2_ref, o_ref):
    idx = idx_ref[...]
    pio = lax.broadcasted_iota(jnp.int32, (_BQ, K, N_IN), 2)
    p = (idx[:, :, None] == pio).astype(jnp.bfloat16)
    g = jnp.dot(p.reshape(_BQ * K, N_IN), hyh_ref[...],
                preferred_element_type=jnp.float32)
    pair = g.reshape(_BQ, K, 2 * HIDDEN) + hx_ref[...][:, None, :]
    act = 0.5 * pair * (1.0 + lax.erf(pair * np.float32(1.0 / np.sqrt(2.0))))
    m = (idx >= 0).astype(jnp.float32)
    summed = jnp.sum(act * m[:, :, None], axis=1)
    cnt = jnp.sum(m, axis=1, keepdims=True)
    res = jnp.dot(summed, w2_ref[...], preferred_element_type=jnp.float32)
    res = res / jnp.maximum(cnt, 1.0)
    o_ref[...] = res + b2_ref[...] * (cnt > 0.0).astype(jnp.float32)


def _mlp(idx, hyh, hx, w2, b2, *, interpret=False):
    grid = (N_Q // _BQ,)
    return pl.pallas_call(
        _mlp_body,
        grid=grid,
        in_specs=[
            pl.BlockSpec((_BQ, K), lambda i: (i, 0)),
            pl.BlockSpec((N_IN, 2 * HIDDEN), lambda i: (0, 0)),
            pl.BlockSpec((_BQ, 2 * HIDDEN), lambda i: (i, 0)),
            pl.BlockSpec((2 * HIDDEN, HIDDEN), lambda i: (0, 0)),
            pl.BlockSpec((1, HIDDEN), lambda i: (0, 0)),
        ],
        out_specs=pl.BlockSpec((_BQ, HIDDEN), lambda i: (i, 0)),
        out_shape=jax.ShapeDtypeStruct((N_Q, HIDDEN), jnp.float32),
        interpret=interpret,
    )(idx, hyh, hx, w2, b2)


_DF = np.repeat(np.arange(NDIM) * 2 * NF, NF) + 2 * np.tile(np.arange(NF), NDIM)
_POS_OUT = NDIM * NF * 2
_PERM = np.concatenate([
    _DF, _DF + 1, _POS_OUT + _DF, _POS_OUT + _DF + 1,
    np.arange(2 * _POS_OUT, 2 * _POS_OUT + 32),
]).astype(np.int32)
_FREQS = ((1.0 / 10000.0) ** (np.arange(NF, dtype=np.float64) / NF)).astype(np.float32)


def kernel(input_feat, input_pos, query_pos, W1, b1, W2, b2):
    y = input_pos[0]
    x = query_pos[0]
    w1p = W1[_PERM, :]
    freqs = jnp.asarray(_FREQS).reshape(1, NF)
    hyh, hx = _prep(y, x, input_feat, w1p, b1.reshape(1, -1), freqs)
    idx = _sc_search(y.reshape(-1), x.reshape(-1))
    return _mlp(idx, hyh, hx, W2, b2.reshape(1, -1))


# speedup vs baseline: 1.6721x; 1.1447x over previous
import functools
import math as _math

import jax
import jax.numpy as jnp
import numpy as np
from jax import lax
from jax.experimental import pallas as pl
from jax.experimental.pallas import tpu as pltpu
from jax.experimental.pallas import tpu_sc as plsc

RADIUS2 = 0.15 * 0.15
NDIM = 3
HIDDEN = 64
NF = 64
N_IN = 1024
N_Q = 512
K = 48
NC = 2
NS = 16
NW = NC * NS
QPW = N_Q // NW
L = 16
NCHUNK = N_IN // L


def _prep_body(ypos_ref, xpos_ref, feat_ref, w1_ref, b1_ref, fi_ref, c_ref,
               hyh_ref, hx_ref):
    fi = fi_ref[...]

    def emb(p):
        z = jnp.broadcast_to(c_ref[9:10, :], p.shape)
        for k in range(8, -1, -1):
            z = z * p + c_ref[k:k + 1, :]
        return z

    acc_y = jnp.dot(feat_ref[...], w1_ref[2 * NDIM * 2 * NF:, :],
                    preferred_element_type=jnp.float32)
    for d in range(NDIM):
        z = emb(ypos_ref[0, :, d:d + 1] * fi)
        acc_y += jnp.dot(z, w1_ref[d * 2 * NF:(d + 1) * 2 * NF, :],
                         preferred_element_type=jnp.float32)
    hyh_ref[...] = acc_y.astype(jnp.bfloat16)
    acc_x = jnp.broadcast_to(b1_ref[...], (N_Q, 2 * HIDDEN))
    for d in range(NDIM):
        z = emb(xpos_ref[0, :, d:d + 1] * fi)
        acc_x = acc_x + jnp.dot(
            z, w1_ref[(NDIM + d) * 2 * NF:(NDIM + d + 1) * 2 * NF, :],
            preferred_element_type=jnp.float32)
    hx_ref[...] = acc_x


def _prep(ypos3, xpos3, feat, w1, b1, fi, coefs, *, interpret=False):
    return pl.pallas_call(
        _prep_body,
        out_shape=(
            jax.ShapeDtypeStruct((N_IN, 2 * HIDDEN), jnp.bfloat16),
            jax.ShapeDtypeStruct((N_Q, 2 * HIDDEN), jnp.float32),
        ),
        interpret=interpret,
    )(ypos3, xpos3, feat, w1, b1, fi, coefs)


def _sc_body(ypos_hbm, xpos_hbm, idx_hbm, yv, qv, idxv):
    wid = lax.axis_index("s") * NC + lax.axis_index("c")
    qbase = wid * QPW
    pltpu.sync_copy(ypos_hbm, yv)
    pltpu.sync_copy(xpos_hbm.at[pl.ds(qbase * NDIM, QPW * NDIM)], qv)
    lanes = lax.iota(jnp.int32, L)
    lanes3 = lanes * NDIM

    def per_query(q, _):
        qi = jnp.full((L,), q, jnp.int32)
        q3 = q * NDIM
        qx = plsc.load_gather(qv, [jnp.full((L,), q3, jnp.int32)])
        qy = plsc.load_gather(qv, [jnp.full((L,), q3 + 1, jnp.int32)])
        qz = plsc.load_gather(qv, [jnp.full((L,), q3 + 2, jnp.int32)])

        for c in range(K // L):
            plsc.store_scatter(idxv, [qi, lanes + c * L],
                               jnp.full((L,), -1, jnp.int32))

        @plsc.parallel_loop(0, NCHUNK, carry=jnp.zeros((L,), jnp.int32),
                            unroll=4)
        def cnt_loop(c, cnt):
            b3 = c * (L * NDIM)
            dx = plsc.load_gather(yv, [lanes3 + b3]) - qx
            dy = plsc.load_gather(yv, [lanes3 + (b3 + 1)]) - qy
            dz = plsc.load_gather(yv, [lanes3 + (b3 + 2)]) - qz
            d2 = dx * dx + dy * dy + dz * dz
            mask = d2 < RADIUS2
            mi = mask.astype(jnp.int32)
            pos = cnt + plsc.cumsum(mi) - 1
            m2 = mask & (pos < K)
            posc = jnp.minimum(jnp.maximum(pos, 0), K - 1)
            plsc.store_scatter(idxv, [qi, posc], lanes + c * L, mask=m2)
            return cnt + plsc.all_reduce_population_count(mask)

        return 0

    lax.fori_loop(0, QPW, per_query, 0)
    pltpu.sync_copy(idxv, idx_hbm.at[pl.ds(qbase, QPW)])


def _sc_search(ypos_flat, xpos_flat):
    mesh = plsc.VectorSubcoreMesh(core_axis_name="c", subcore_axis_name="s")
    k = pl.kernel(
        _sc_body,
        out_type=jax.ShapeDtypeStruct((N_Q, K), jnp.int32),
        mesh=mesh,
        compiler_params=pltpu.CompilerParams(needs_layout_passes=False),
        scratch_types=[
            pltpu.VMEM((NDIM * N_IN,), jnp.float32),
            pltpu.VMEM((NDIM * QPW,), jnp.float32),
            pltpu.VMEM((QPW, K), jnp.int32),
        ],
    )
    return k(ypos_flat, xpos_flat)


_BQ = 64


def _mlp_body(idx_ref, hyh_ref, hx_ref, w2_ref, b2_ref, o_ref):
    idx = idx_ref[...]
    pio = lax.broadcasted_iota(jnp.int32, (_BQ, K, N_IN), 2)
    p = (idx[:, :, None] == pio).astype(jnp.bfloat16)
    g = jnp.dot(p.reshape(_BQ * K, N_IN), hyh_ref[...],
                preferred_element_type=jnp.float32)
    pair = g.reshape(_BQ, K, 2 * HIDDEN) + hx_ref[...][:, None, :]
    act = 0.5 * pair * (1.0 + lax.erf(pair * np.float32(1.0 / np.sqrt(2.0))))
    m = (idx >= 0).astype(jnp.float32)
    summed = jnp.sum(act * m[:, :, None], axis=1)
    cnt = jnp.sum(m, axis=1, keepdims=True)
    res = jnp.dot(summed, w2_ref[...], preferred_element_type=jnp.float32)
    res = res / jnp.maximum(cnt, 1.0)
    o_ref[...] = res + b2_ref[...] * (cnt > 0.0).astype(jnp.float32)


def _mlp(idx, hyh, hx, w2, b2, *, interpret=False):
    grid = (N_Q // _BQ,)
    return pl.pallas_call(
        _mlp_body,
        grid=grid,
        in_specs=[
            pl.BlockSpec((_BQ, K), lambda i: (i, 0)),
            pl.BlockSpec((N_IN, 2 * HIDDEN), lambda i: (0, 0)),
            pl.BlockSpec((_BQ, 2 * HIDDEN), lambda i: (i, 0)),
            pl.BlockSpec((2 * HIDDEN, HIDDEN), lambda i: (0, 0)),
            pl.BlockSpec((1, HIDDEN), lambda i: (0, 0)),
        ],
        out_specs=pl.BlockSpec((_BQ, HIDDEN), lambda i: (i, 0)),
        out_shape=jax.ShapeDtypeStruct((N_Q, HIDDEN), jnp.float32),
        interpret=interpret,
    )(idx, hyh, hx, w2, b2)


_FREQS = ((1.0 / 10000.0) ** (np.arange(NF, dtype=np.float64) / NF)).astype(np.float32)
_FREQS_I = np.repeat(_FREQS, 2).reshape(1, 2 * NF)
_COEFS = np.zeros((16, 2 * NF), np.float32)
for _k in range(10):
    if _k % 2 == 1:
        _COEFS[_k, 0::2] = (-1.0) ** ((_k - 1) // 2) / float(_math.factorial(_k))
    else:
        _COEFS[_k, 1::2] = (-1.0) ** (_k // 2) / float(_math.factorial(_k))


def kernel(input_feat, input_pos, query_pos, W1, b1, W2, b2):
    fi = jnp.asarray(_FREQS_I)
    coefs = jnp.asarray(_COEFS)
    hyh, hx = _prep(input_pos, query_pos, input_feat, W1,
                    b1.reshape(1, -1), fi, coefs)
    idx = _sc_search(input_pos.reshape(-1), query_pos.reshape(-1))
    return _mlp(idx, hyh, hx, W2, b2.reshape(1, -1))
